# Initial kernel scaffold; baseline (speedup 1.0000x reference)
import jax
import jax.numpy as jnp
from jax.experimental import pallas as pl


def kernel(x_user, x_item, edge_index_e0, edge_index_e1, W_e0, b_e0, W_e1, b_e1):
    raise NotImplementedError("write your pallas kernel here")


# speedup vs baseline: 2.8076x; 2.8076x over previous
import functools

import jax
import jax.numpy as jnp
from jax import lax
from jax.experimental import pallas as pl
from jax.experimental.pallas import tpu as pltpu
from jax.experimental.pallas import tpu_sc as plsc

N_NODE = 50000
E = 320000
D_IN = 128
D_OUT = 64

NC = 2
NS = 16
L = 16

CHUNK = 128
BI = 16
NBI = 10
J = BI * NBI
EPT = J * CHUNK
E_PAD = NS * EPT

HALF = N_NODE // NC
ROWS_PT = 1568
H_ACC = NS * ROWS_PT
GARBAGE = 25080
CNT_W = 8
OUT_PT = 1568
PAD_DST = 1 << 29


def _mm_body(x_ref, w_ref, b_ref, o_ref):
    o_ref[...] = (
        jnp.dot(x_ref[...], w_ref[...], preferred_element_type=jnp.float32)
        + b_ref[...]
    )


@jax.jit
def _linear(x, W, b):
    blk = 1000
    return pl.pallas_call(
        _mm_body,
        grid=(N_NODE // blk,),
        in_specs=[
            pl.BlockSpec((blk, D_IN), lambda i: (i, 0)),
            pl.BlockSpec((D_IN, D_OUT), lambda i: (0, 0)),
            pl.BlockSpec((1, D_OUT), lambda i: (0, 0)),
        ],
        out_specs=pl.BlockSpec((blk, D_OUT), lambda i: (i, 0)),
        out_shape=jax.ShapeDtypeStruct((N_NODE, D_OUT), jnp.float32),
    )(x, W, b.reshape(1, D_OUT))


def _sc_body(wh_hbm, src_hbm, dst_hbm, ones_hbm, zeros_hbm,
             sum_out, cnt_out,
             src_blk, dstl_blk, rows_v, ones_v, acc_sh, cnt_sh, sem):
    c = lax.axis_index("c")
    s = lax.axis_index("s")

    zv = jnp.zeros((L,), jnp.float32)

    @pl.loop(0, CHUNK)
    def _(r):
        for k in range(D_OUT // L):
            rows_v[r, pl.ds(k * L, L)] = zv

    pltpu.sync_copy(ones_hbm, ones_v)

    a0 = s * ROWS_PT
    for k in range(ROWS_PT // CHUNK):
        pltpu.sync_copy(rows_v, acc_sh.at[pl.ds(a0 + k * CHUNK, CHUNK)])
        pltpu.sync_copy(zeros_hbm, cnt_sh.at[pl.ds(a0 + k * CHUNK, CHUNK)])
    tail = ROWS_PT % CHUNK
    if tail:
        t0 = a0 + ROWS_PT - tail
        pltpu.sync_copy(rows_v.at[pl.ds(0, tail)], acc_sh.at[pl.ds(t0, tail)])
        pltpu.sync_copy(zeros_hbm.at[pl.ds(0, tail)],
                        cnt_sh.at[pl.ds(t0, tail)])

    plsc.subcore_barrier()

    base = jnp.full((L,), c * HALF, jnp.int32)
    garbage = jnp.full((L,), GARBAGE, jnp.int32)

    @pl.loop(0, NBI)
    def _(bi):
        pltpu.sync_copy(src_hbm.at[s, pl.ds(bi * BI, BI)], src_blk)
        pltpu.sync_copy(dst_hbm.at[s, pl.ds(bi * BI, BI)], dstl_blk)

        @pl.loop(0, BI)
        def _(r):
            for k in range(CHUNK // L):
                d = dstl_blk[r, pl.ds(k * L, L)]
                loc = d - base
                ok = (loc >= 0) & (loc < HALF)
                dstl_blk[r, pl.ds(k * L, L)] = jnp.where(ok, loc, garbage)

        @pl.loop(0, BI)
        def _(j):
            pltpu.async_copy(wh_hbm.at[src_blk.at[j]], rows_v, sem).wait()
            pltpu.sync_copy(rows_v, acc_sh.at[dstl_blk.at[j]], add=True)
            pltpu.sync_copy(ones_v, cnt_sh.at[dstl_blk.at[j]], add=True)

    plsc.subcore_barrier()

    o0 = c * HALF

    @pl.when(s < NS - 1)
    def _():
        pltpu.sync_copy(acc_sh.at[pl.ds(s * OUT_PT, OUT_PT)],
                        sum_out.at[pl.ds(o0 + s * OUT_PT, OUT_PT)])
        pltpu.sync_copy(cnt_sh.at[pl.ds(s * OUT_PT, OUT_PT)],
                        cnt_out.at[pl.ds(o0 + s * OUT_PT, OUT_PT)])

    @pl.when(s == NS - 1)
    def _():
        tail0 = (NS - 1) * OUT_PT
        tail_n = HALF - tail0
        pltpu.sync_copy(acc_sh.at[pl.ds(tail0, tail_n)],
                        sum_out.at[pl.ds(o0 + tail0, tail_n)])
        pltpu.sync_copy(cnt_sh.at[pl.ds(tail0, tail_n)],
                        cnt_out.at[pl.ds(o0 + tail0, tail_n)])


_sc_aggregate = pl.kernel(
    _sc_body,
    out_type=[
        jax.ShapeDtypeStruct((N_NODE, D_OUT), jnp.float32),
        jax.ShapeDtypeStruct((N_NODE, CNT_W), jnp.float32),
    ],
    mesh=plsc.VectorSubcoreMesh(
        core_axis_name="c", subcore_axis_name="s",
        num_cores=NC, num_subcores=NS,
    ),
    compiler_params=pltpu.CompilerParams(use_tc_tiling_on_sc=False),
    scratch_types=[
        pltpu.VMEM((BI, CHUNK), jnp.int32),
        pltpu.VMEM((BI, CHUNK), jnp.int32),
        pltpu.VMEM((CHUNK, D_OUT), jnp.float32),
        pltpu.VMEM((CHUNK, CNT_W), jnp.float32),
        pltpu.VMEM_SHARED((H_ACC, D_OUT), jnp.float32),
        pltpu.VMEM_SHARED((H_ACC, CNT_W), jnp.float32),
        pltpu.SemaphoreType.DMA,
    ],
)


def _div_body(s_ref, c_ref, o_ref):
    cnt = jnp.maximum(c_ref[:, 0:1], 1.0)
    o_ref[...] = s_ref[...] / cnt


@jax.jit
def _mean_div(summed, counts):
    blk = 1000
    return pl.pallas_call(
        _div_body,
        grid=(N_NODE // blk,),
        in_specs=[
            pl.BlockSpec((blk, D_OUT), lambda i: (i, 0)),
            pl.BlockSpec((blk, CNT_W), lambda i: (i, 0)),
        ],
        out_specs=pl.BlockSpec((blk, D_OUT), lambda i: (i, 0)),
        out_shape=jax.ShapeDtypeStruct((N_NODE, D_OUT), jnp.float32),
    )(summed, counts)


def _pad_edges(edge_index):
    src = edge_index[0].astype(jnp.int32)
    dst = edge_index[1].astype(jnp.int32)
    pad = E_PAD - E
    src = jnp.concatenate([src, jnp.zeros((pad,), jnp.int32)])
    dst = jnp.concatenate([dst, jnp.full((pad,), PAD_DST, jnp.int32)])
    return src.reshape(NS, J, CHUNK), dst.reshape(NS, J, CHUNK)


def kernel(x_user, x_item, edge_index_e0, edge_index_e1, W_e0, b_e0, W_e1, b_e1):
    ones8 = jnp.ones((CHUNK, CNT_W), jnp.float32)
    zeros8 = jnp.zeros((CHUNK, CNT_W), jnp.float32)

    Wh_user = _linear(x_user, W_e0, b_e0)
    Wh_item = _linear(x_item, W_e1, b_e1)

    src0, dst0 = _pad_edges(edge_index_e0)
    src1, dst1 = _pad_edges(edge_index_e1)

    sum_item, cnt_item = _sc_aggregate(Wh_user, src0, dst0, ones8, zeros8)
    sum_user, cnt_user = _sc_aggregate(Wh_item, src1, dst1, ones8, zeros8)

    h_item = _mean_div(sum_item, cnt_item)
    h_user = _mean_div(sum_user, cnt_user)
    return (h_user, h_item)
